# baseline (speedup 1.0000x reference)
import jax
import jax.numpy as jnp
import numpy as np
from jax.experimental import pallas as pl

B, P, N = 4, 16, 4096
S = 16
CH = [2, 16, 32, 64, 64]
EPS = 1e-5


def _lin_kernel(x_ref, w_ref, o_ref):
    o_ref[...] = jnp.dot(x_ref[...], w_ref[...], preferred_element_type=jnp.float32)


def kernel(parts_voxels, parts_labels, w0, w1, w2, w3, g0, g1, g2, g3, W_lin):
    BP = B * P
    coords = parts_voxels.reshape(BP * N, 3)
    labels = parts_labels.reshape(BP * N).astype(jnp.float32)
    feats = jnp.stack([jnp.ones_like(labels), labels], axis=-1)
    bi = jnp.repeat(jnp.arange(BP), N)
    dense = jnp.zeros((BP, S, S, S, 2), jnp.float32).at[bi, coords[:, 0], coords[:, 1], coords[:, 2]].add(feats)
    x = dense
    for w, g in ((w0, g0), (w1, g1), (w2, g2), (w3, g3)):
        x = jax.lax.conv_general_dilated(
            x, w, window_strides=(2, 2, 2), padding=[(1, 1)] * 3,
            dimension_numbers=("NDHWC", "DHWIO", "NDHWC"))
        x = x * (g / jnp.sqrt(1.0 + EPS))
        x = jax.nn.relu(x)
    x = x.reshape(BP, CH[-1])
    x = pl.pallas_call(
        _lin_kernel,
        out_shape=jax.ShapeDtypeStruct((BP, CH[-1]), jnp.float32),
    )(x, W_lin.T)
    return x.reshape(B, P, CH[-1])


# speedup vs baseline: 2.5217x; 2.5217x over previous
import functools

import jax
import jax.numpy as jnp
import numpy as np
from jax import lax
from jax.experimental import pallas as pl

B, P, N = 4, 16, 4096
S = 16
BP = B * P
CH = [2, 16, 32, 64, 64]
EPS = 1e-5
PACK = 1 << 18
_BN = float(1.0 / np.sqrt(1.0 + EPS))


def _bigw(w_eff, dx, y_dim):
    oy_dim = y_dim // 2
    ci, co = w_eff.shape[3], w_eff.shape[4]
    yy = np.arange(y_dim)[:, None]
    oo = np.arange(oy_dim)[None, :]
    d = yy - 2 * oo + 1
    valid = jnp.asarray((d >= 0) & (d <= 2), jnp.float32)
    dc = np.clip(d, 0, 2)
    g = w_eff[dx][dc]
    g = g[:, :, dc]
    g = g * valid[:, :, None, None, None, None]
    g = g * valid[None, None, :, :, None, None]
    g = g.transpose(4, 0, 2, 5, 1, 3)
    return g.reshape(ci * y_dim * y_dim, co * oy_dim * oy_dim)


def _xsel(x3, lanes):
    ox = x3.shape[1]
    even = x3[:, :, :lanes]
    odd = x3[:, :, lanes:]
    if ox > 1:
        shifted = jnp.concatenate(
            [jnp.zeros_like(odd[:, :1]), odd[:, :ox - 1]], axis=1)
    else:
        shifted = jnp.zeros_like(odd)
    m = BP * ox
    return (shifted.reshape(m, lanes), even.reshape(m, lanes),
            odd.reshape(m, lanes))


def _mm(a, b):
    return lax.dot_general(a, b, (((1,), (0,)), ((), ())),
                           preferred_element_type=jnp.float32)


def _encode_body(d_ref, bw1c_ref, bw1l_ref, bw2_ref, bw3_ref, bw4_ref,
                 wl_ref, o_ref):
    d = d_ref[...]
    cnt = (d >> 18).astype(jnp.float32)
    lbl = (d & (PACK - 1)).astype(jnp.float32)

    ac = _xsel(cnt.reshape(BP, 8, 512), 256)
    al = _xsel(lbl.reshape(BP, 8, 512), 256)
    y1 = _mm(ac[0], bw1c_ref[0]) + _mm(ac[1], bw1c_ref[1]) + _mm(ac[2], bw1c_ref[2])
    y1 += _mm(al[0], bw1l_ref[0]) + _mm(al[1], bw1l_ref[1]) + _mm(al[2], bw1l_ref[2])
    x = jnp.maximum(y1, 0.0).reshape(BP, 8, 1024)

    a = _xsel(x.reshape(BP, 4, 2048), 1024)
    y = _mm(a[0], bw2_ref[0]) + _mm(a[1], bw2_ref[1]) + _mm(a[2], bw2_ref[2])
    x = jnp.maximum(y, 0.0).reshape(BP, 4, 512)

    a = _xsel(x.reshape(BP, 2, 1024), 512)
    y = _mm(a[0], bw3_ref[0]) + _mm(a[1], bw3_ref[1]) + _mm(a[2], bw3_ref[2])
    x = jnp.maximum(y, 0.0).reshape(BP, 2, 256)

    a = _xsel(x.reshape(BP, 1, 512), 256)
    y = _mm(a[1], bw4_ref[0]) + _mm(a[2], bw4_ref[1])
    x = jnp.maximum(y, 0.0)

    o_ref[...] = lax.dot_general(x, wl_ref[...], (((1,), (1,)), ((), ())),
                                 preferred_element_type=jnp.float32)


@functools.partial(jax.jit, static_argnames=("interpret",))
def _encode(dense_i32, bw1c, bw1l, bw2, bw3, bw4, W_lin, interpret=False):
    return pl.pallas_call(
        _encode_body,
        out_shape=jax.ShapeDtypeStruct((BP, CH[4]), jnp.float32),
        interpret=interpret,
    )(dense_i32, bw1c, bw1l, bw2, bw3, bw4, W_lin)


def _prep_weights(w0, w1, w2, w3, g0, g1, g2, g3):
    w0e = w0 * (g0 * _BN)
    w1e = w1 * (g1 * _BN)
    w2e = w2 * (g2 * _BN)
    w3e = w3 * (g3 * _BN)
    w0c = w0e[:, :, :, 0:1]
    w0l = w0e[:, :, :, 1:2]
    bw1c = jnp.stack([_bigw(w0c, dx, 16) for dx in range(3)])
    bw1l = jnp.stack([_bigw(w0l, dx, 16) for dx in range(3)])
    bw2 = jnp.stack([_bigw(w1e, dx, 8) for dx in range(3)])
    bw3 = jnp.stack([_bigw(w2e, dx, 4) for dx in range(3)])
    bw4 = jnp.stack([_bigw(w3e, dx, 2) for dx in (1, 2)])
    return bw1c, bw1l, bw2, bw3, bw4


def kernel(parts_voxels, parts_labels, w0, w1, w2, w3, g0, g1, g2, g3, W_lin):
    coords = parts_voxels.reshape(BP * N, 3)
    bi = jnp.repeat(jnp.arange(BP, dtype=jnp.int32), N)
    gidx = ((bi * S + coords[:, 0]) * S + coords[:, 1]) * S + coords[:, 2]
    val = parts_labels.reshape(BP * N) + PACK
    dense = jnp.zeros((BP * S * S * S,), jnp.int32).at[gidx].add(val)
    dense = dense.reshape(BP, S, S * S)

    bw1c, bw1l, bw2, bw3, bw4 = _prep_weights(w0, w1, w2, w3, g0, g1, g2, g3)
    out = _encode(dense, bw1c, bw1l, bw2, bw3, bw4, W_lin)
    return out.reshape(B, P, CH[4])


# speedup vs baseline: 2.5761x; 1.0216x over previous
import functools

import jax
import jax.numpy as jnp
import numpy as np
from jax import lax
from jax.experimental import pallas as pl
from jax.experimental.pallas import tpu as pltpu
from jax.experimental.pallas import tpu_sc as plsc

B, P, N = 4, 16, 4096
S = 16
BP = B * P
CH = [2, 16, 32, 64, 64]
EPS = 1e-5
PACK = 1 << 18
_BN = float(1.0 / np.sqrt(1.0 + EPS))


def _bigw(w_eff, dx, y_dim):
    oy_dim = y_dim // 2
    ci, co = w_eff.shape[3], w_eff.shape[4]
    yy = np.arange(y_dim)[:, None]
    oo = np.arange(oy_dim)[None, :]
    d = yy - 2 * oo + 1
    valid = jnp.asarray((d >= 0) & (d <= 2), jnp.float32)
    dc = np.clip(d, 0, 2)
    g = w_eff[dx][dc]
    g = g[:, :, dc]
    g = g * valid[:, :, None, None, None, None]
    g = g * valid[None, None, :, :, None, None]
    g = g.transpose(4, 0, 2, 5, 1, 3)
    return g.reshape(ci * y_dim * y_dim, co * oy_dim * oy_dim)


def _xsel(x3, lanes):
    ox = x3.shape[1]
    even = x3[:, :, :lanes]
    odd = x3[:, :, lanes:]
    if ox > 1:
        shifted = jnp.concatenate(
            [jnp.zeros_like(odd[:, :1]), odd[:, :ox - 1]], axis=1)
    else:
        shifted = jnp.zeros_like(odd)
    m = BP * ox
    return (shifted.reshape(m, lanes), even.reshape(m, lanes),
            odd.reshape(m, lanes))


def _mm(a, b):
    return lax.dot_general(a, b, (((1,), (0,)), ((), ())),
                           preferred_element_type=jnp.float32)


def _encode_body(d_ref, bw1c_ref, bw1l_ref, bw2_ref, bw3_ref, bw4_ref,
                 wl_ref, o_ref):
    d = d_ref[...]
    cnt = (d >> 18).astype(jnp.float32)
    lbl = (d & (PACK - 1)).astype(jnp.float32)

    ac = _xsel(cnt.reshape(BP, 8, 512), 256)
    al = _xsel(lbl.reshape(BP, 8, 512), 256)
    y1 = _mm(ac[0], bw1c_ref[0]) + _mm(ac[1], bw1c_ref[1]) + _mm(ac[2], bw1c_ref[2])
    y1 += _mm(al[0], bw1l_ref[0]) + _mm(al[1], bw1l_ref[1]) + _mm(al[2], bw1l_ref[2])
    x = jnp.maximum(y1, 0.0).reshape(BP, 8, 1024)

    a = _xsel(x.reshape(BP, 4, 2048), 1024)
    y = _mm(a[0], bw2_ref[0]) + _mm(a[1], bw2_ref[1]) + _mm(a[2], bw2_ref[2])
    x = jnp.maximum(y, 0.0).reshape(BP, 4, 512)

    a = _xsel(x.reshape(BP, 2, 1024), 512)
    y = _mm(a[0], bw3_ref[0]) + _mm(a[1], bw3_ref[1]) + _mm(a[2], bw3_ref[2])
    x = jnp.maximum(y, 0.0).reshape(BP, 2, 256)

    a = _xsel(x.reshape(BP, 1, 512), 256)
    y = _mm(a[1], bw4_ref[0]) + _mm(a[2], bw4_ref[1])
    x = jnp.maximum(y, 0.0)

    o_ref[...] = lax.dot_general(x, wl_ref[...], (((1,), (1,)), ((), ())),
                                 preferred_element_type=jnp.float32)


@functools.partial(jax.jit, static_argnames=("interpret",))
def _encode(dense_i32, bw1c, bw1l, bw2, bw3, bw4, W_lin, interpret=False):
    return pl.pallas_call(
        _encode_body,
        out_shape=jax.ShapeDtypeStruct((BP, CH[4]), jnp.float32),
        interpret=interpret,
    )(dense_i32, bw1c, bw1l, bw2, bw3, bw4, W_lin)


_SC_MESH = plsc.VectorSubcoreMesh(core_axis_name="c", subcore_axis_name="s")


@functools.partial(
    pl.kernel,
    mesh=_SC_MESH,
    out_type=jax.ShapeDtypeStruct((BP * N,), jnp.int32),
    scratch_types=[
        pltpu.VMEM((N,), jnp.int32),
        pltpu.VMEM((N,), jnp.int32),
        pltpu.VMEM((N,), jnp.int32),
        pltpu.VMEM((N,), jnp.int32),
        pltpu.VMEM((32, 128), jnp.int32),
        pltpu.VMEM((32, 128), jnp.int32),
        pltpu.VMEM((2 * N,), jnp.int32),
        pltpu.VMEM_SHARED((16 * 2 * N,), jnp.int32),
    ],
)
def _sc_scatter(xs_h, ys_h, zs_h, lb_h, out_h, xv, yv, zv, lv, iv, vv, zbuf,
                shared):
    cid = lax.axis_index("c")
    sid = lax.axis_index("s")
    wid = cid * 16 + sid
    reg = sid * (2 * N)

    def zbody(i, carry):
        zbuf[pl.ds(i * 16, 16)] = jnp.zeros((16,), jnp.int32)
        return carry

    lax.fori_loop(0, 512, zbody, 0)
    pltpu.sync_copy(zbuf, shared.at[pl.ds(reg, 2 * N)])

    for j in range(2):
        base = (wid * 2 + j) * N
        pltpu.sync_copy(xs_h.at[pl.ds(base, N)], xv)
        pltpu.sync_copy(ys_h.at[pl.ds(base, N)], yv)
        pltpu.sync_copy(zs_h.at[pl.ds(base, N)], zv)
        pltpu.sync_copy(lb_h.at[pl.ds(base, N)], lv)

        def cbody(k, carry):
            for u in range(8):
                o = (k * 8 + u) * 16
                gi = xv[pl.ds(o, 16)] * 256 + yv[pl.ds(o, 16)] * 16 \
                    + zv[pl.ds(o, 16)] + (reg + j * N)
                iv[k, pl.ds(u * 16, 16)] = gi
                vv[k, pl.ds(u * 16, 16)] = lv[pl.ds(o, 16)] + PACK
            pltpu.sync_copy(vv.at[k], shared.at[iv.at[k]], add=True)
            return carry

        lax.fori_loop(0, 32, cbody, 0)

    pltpu.sync_copy(shared.at[pl.ds(reg, 2 * N)],
                    out_h.at[pl.ds(wid * 2 * N, 2 * N)])


def _prep_weights(w0, w1, w2, w3, g0, g1, g2, g3):
    w0e = w0 * (g0 * _BN)
    w1e = w1 * (g1 * _BN)
    w2e = w2 * (g2 * _BN)
    w3e = w3 * (g3 * _BN)
    w0c = w0e[:, :, :, 0:1]
    w0l = w0e[:, :, :, 1:2]
    bw1c = jnp.stack([_bigw(w0c, dx, 16) for dx in range(3)])
    bw1l = jnp.stack([_bigw(w0l, dx, 16) for dx in range(3)])
    bw2 = jnp.stack([_bigw(w1e, dx, 8) for dx in range(3)])
    bw3 = jnp.stack([_bigw(w2e, dx, 4) for dx in range(3)])
    bw4 = jnp.stack([_bigw(w3e, dx, 2) for dx in (1, 2)])
    return bw1c, bw1l, bw2, bw3, bw4


def kernel(parts_voxels, parts_labels, w0, w1, w2, w3, g0, g1, g2, g3, W_lin):
    coords = parts_voxels.reshape(BP * N, 3)
    xs = coords[:, 0]
    ys = coords[:, 1]
    zs = coords[:, 2]
    lb = parts_labels.reshape(BP * N)
    dense = _sc_scatter(xs, ys, zs, lb)
    dense = dense.reshape(BP, S, S * S)

    bw1c, bw1l, bw2, bw3, bw4 = _prep_weights(w0, w1, w2, w3, g0, g1, g2, g3)
    out = _encode(dense, bw1c, bw1l, bw2, bw3, bw4, W_lin)
    return out.reshape(B, P, CH[4])
